# baseline (speedup 1.0000x reference)
import functools
import math

import jax
import jax.numpy as jnp
from jax import lax
from jax.experimental import pallas as pl
from jax.experimental.pallas import tpu as pltpu
from jax.experimental.pallas import tpu_sc as plsc

D_MODEL = 64
VOCAB = 1000000
VPAD = 1000064
NVB = VPAD // 128
NPAIR = NVB * 64
B1 = 4096
B2 = 200
NW = 32
K = 128
NCHUNK = B1 * B2 // K
PER_W = NCHUNK // NW
BLK_PER_W = -(-NVB // NW)
NBUF = 2
RS = 129
SCALE = math.sqrt(D_MODEL)

_mesh = plsc.VectorSubcoreMesh(core_axis_name="c", subcore_axis_name="s")
_params = pltpu.CompilerParams(
    use_tc_tiling_on_sc=True,
    needs_layout_passes=False,
    disable_bounds_checks=True,
)


@functools.partial(
    pl.kernel,
    mesh=_mesh,
    out_type=jax.ShapeDtypeStruct((NVB, D_MODEL, 128), jnp.float32),
    compiler_params=_params,
    scratch_types=[
        pltpu.VMEM((NBUF, D_MODEL, RS), jnp.float32),
        pltpu.VMEM((NBUF, D_MODEL, 128), jnp.float32),
        pltpu.SemaphoreType.DMA((NBUF,)),
        pltpu.SemaphoreType.DMA((NBUF,)),
    ],
)
def _relayout(lutT_hbm, scr_hbm, blk_v, pair_v, rsem, wsem):
    wid = lax.axis_index("s") * 2 + lax.axis_index("c")

    def blk_id(j):
        return wid + j * NW

    def src_slice(j):
        return lutT_hbm.at[:, pl.ds(blk_id(j) * K, K)]

    def blk_dst(b):
        return blk_v.at[b, :, pl.ds(0, K)]

    def start_read(j, b):
        pltpu.async_copy(src_slice(j), blk_dst(b), rsem.at[b])

    def wait_read(j, b):
        pltpu.make_async_copy(src_slice(j), blk_dst(b), rsem.at[b]).wait()

    def start_write(j, b):
        pltpu.async_copy(pair_v.at[b], scr_hbm.at[blk_id(j)], wsem.at[b])

    def wait_write(j, b):
        pltpu.make_async_copy(
            pair_v.at[b], scr_hbm.at[blk_id(j)], wsem.at[b]
        ).wait()

    lanes = lax.iota(jnp.int32, 16)
    feat_idx = [lanes + c0 % D_MODEL for c0 in range(0, 128, 16)]

    def transpose_block(b):
        def pbody(p, carry):
            for ci, c0 in enumerate(range(0, 128, 16)):
                voc = jnp.full((16,), 2 * p + c0 // D_MODEL, jnp.int32)
                v = plsc.load_gather(blk_v.at[b], [feat_idx[ci], voc])
                pair_v[b, p, pl.ds(c0, 16)] = v
            return carry

        lax.fori_loop(0, D_MODEL, pbody, 0, unroll=2)

    def in_range(j):
        return blk_id(j) < NVB

    for b in range(NBUF):
        @pl.when(in_range(b))
        def _():
            start_read(b, b)

    def outer(grp, carry):
        for b in range(NBUF):
            j = grp * NBUF + b

            @pl.when(in_range(j))
            def _():
                wait_read(j, b)

                @pl.when(j >= NBUF)
                def _():
                    wait_write(j - NBUF, b)

                transpose_block(b)
                start_write(j, b)

                @pl.when(in_range(j + NBUF))
                def _():
                    start_read(j + NBUF, b)

        return carry

    lax.fori_loop(0, -(-BLK_PER_W // NBUF), outer, 0)

    def drain(j, carry):
        @pl.when(in_range(j) & jnp.logical_not(in_range(j + NBUF)))
        def _():
            wait_write(j, j % NBUF)
        return carry

    lax.fori_loop(BLK_PER_W - 2 * NBUF, BLK_PER_W + NBUF, drain, 0)


@functools.partial(
    pl.kernel,
    mesh=_mesh,
    out_type=jax.ShapeDtypeStruct((B2, D_MODEL, B1), jnp.float32),
    compiler_params=_params,
    scratch_types=[
        pltpu.VMEM((PER_W * K,), jnp.int32),
        pltpu.VMEM((NBUF, K), jnp.int32),
        pltpu.VMEM((NBUF, K), jnp.int32),
        pltpu.VMEM((NBUF, K, RS), jnp.float32),
        pltpu.VMEM((NBUF, D_MODEL, 128), jnp.float32),
        pltpu.SemaphoreType.DMA((NBUF,)),
        pltpu.SemaphoreType.DMA((NBUF,)),
    ],
)
def _embed(x_hbm, scr_hbm, out_hbm, idx_v, idxh_v, par_v, rows_v, tbuf_v,
           gsem, wsem):
    wid = lax.axis_index("s") * 2 + lax.axis_index("c")
    cid0 = wid * PER_W

    pltpu.sync_copy(x_hbm.at[wid], idx_v)

    def prep(t, b):
        for g in range(K // 16):
            v = idx_v[pl.ds(t * K + g * 16, 16)]
            sl = pl.ds(g * 16, 16)
            idxh_v[b, sl] = lax.shift_right_logical(v, 1)
            par_v[b, sl] = lax.bitwise_and(v, 1) * D_MODEL


    def rows_dst(b):
        return rows_v.at[b, :, pl.ds(0, 128)]

    def start_gather(b):
        pltpu.async_copy(scr_hbm.at[idxh_v.at[b]], rows_dst(b), gsem.at[b])

    def wait_gather(b):
        pltpu.make_async_copy(
            scr_hbm.at[idxh_v.at[b]], rows_dst(b), gsem.at[b]
        ).wait()

    def out_slice(t):
        cid = cid0 + t
        b2 = lax.shift_right_logical(cid, 5)
        bh = lax.bitwise_and(cid, 31)
        return out_hbm.at[b2, :, pl.ds(bh * K, K)]

    def start_write(t, b):
        pltpu.async_copy(tbuf_v.at[b], out_slice(t), wsem.at[b])

    def wait_write(t, b):
        pltpu.make_async_copy(tbuf_v.at[b], out_slice(t), wsem.at[b]).wait()

    lanes = lax.iota(jnp.int32, 16)
    rowsel = [lanes + g * 16 for g in range(K // 16)]

    def transpose_scale(t, b):
        colbase = [par_v[b, pl.ds(g * 16, 16)] for g in range(K // 16)]

        def dbody(d, carry):
            for g in range(K // 16):
                v = plsc.load_gather(rows_v.at[b], [rowsel[g], colbase[g] + d])
                tbuf_v[b, d, pl.ds(g * 16, 16)] = v * SCALE
            return carry

        lax.fori_loop(0, D_MODEL, dbody, 0, unroll=2)

    for b in range(NBUF):
        prep(b, b)
        start_gather(b)

    def outer(grp, carry):
        for b in range(NBUF):
            t = grp * NBUF + b
            wait_gather(b)

            @pl.when(t >= NBUF)
            def _():
                wait_write(t - NBUF, b)

            transpose_scale(t, b)
            start_write(t, b)

            @pl.when(t + NBUF < PER_W)
            def _():
                prep(t + NBUF, b)
                start_gather(b)

        return carry

    lax.fori_loop(0, PER_W // NBUF, outer, 0)

    for b in range(NBUF):
        wait_write(PER_W - NBUF + b, b)


@jax.jit
def kernel(x, lut):
    lutT = lut.T
    scr = _relayout(lutT)
    xw = x.T.reshape(NW, PER_W * K)
    out5 = _embed(xw, scr.reshape(NPAIR, 128))
    return out5.transpose(2, 0, 1)


# speedup vs baseline: 1.5642x; 1.5642x over previous
import functools
import math

import jax
import jax.numpy as jnp
from jax import lax
from jax.experimental import pallas as pl
from jax.experimental.pallas import tpu as pltpu
from jax.experimental.pallas import tpu_sc as plsc

D_MODEL = 64
VOCAB = 1000000
B = 4096 * 200
NW = 32
K = 128
PER_W = B // NW // K
RB = 160
NRB = VOCAB // RB
RB_PER_W = -(-NRB // NW)
NBUF = 2
SCALE = math.sqrt(D_MODEL)

_mesh = plsc.VectorSubcoreMesh(core_axis_name="c", subcore_axis_name="s")
_tiled = pltpu.CompilerParams(
    use_tc_tiling_on_sc=True,
    needs_layout_passes=False,
    disable_bounds_checks=True,
)
_linear = pltpu.CompilerParams(
    needs_layout_passes=False,
    disable_bounds_checks=True,
)


@functools.partial(
    pl.kernel,
    mesh=_mesh,
    out_type=jax.ShapeDtypeStruct((VOCAB // 2, 128), jnp.float32),
    compiler_params=_tiled,
    scratch_types=[
        pltpu.VMEM((NBUF, RB, D_MODEL), jnp.float32),
        pltpu.VMEM((NBUF, RB // 2, 128), jnp.float32),
        pltpu.SemaphoreType.DMA((NBUF,)),
        pltpu.SemaphoreType.DMA((NBUF,)),
    ],
)
def _repack(lut_hbm, scr_hbm, in_v, out_v, rsem, wsem):
    wid = lax.axis_index("s") * 2 + lax.axis_index("c")

    def blk(j):
        return wid + j * NW

    def src_slice(j):
        return lut_hbm.at[pl.ds(blk(j) * RB, RB), :]

    def dst_slice(j):
        return scr_hbm.at[pl.ds(blk(j) * (RB // 2), RB // 2), :]

    def start_read(j, b):
        pltpu.async_copy(src_slice(j), in_v.at[b], rsem.at[b])

    def wait_read(j, b):
        pltpu.make_async_copy(src_slice(j), in_v.at[b], rsem.at[b]).wait()

    def start_write(j, b):
        pltpu.async_copy(out_v.at[b], dst_slice(j), wsem.at[b])

    def wait_write(j, b):
        pltpu.make_async_copy(out_v.at[b], dst_slice(j), wsem.at[b]).wait()

    def pack(b):
        def pbody(p, carry):
            for q in range(D_MODEL // 16):
                sl = pl.ds(q * 16, 16)
                out_v[b, p, pl.ds(q * 16, 16)] = in_v[b, 2 * p, sl]
                out_v[b, p, pl.ds(64 + q * 16, 16)] = in_v[b, 2 * p + 1, sl]
            return carry

        lax.fori_loop(0, RB // 2, pbody, 0, unroll=2)

    def in_range(j):
        return blk(j) < NRB

    for b in range(NBUF):
        @pl.when(in_range(b))
        def _():
            start_read(b, b)

    def outer(grp, carry):
        for b in range(NBUF):
            j = grp * NBUF + b

            @pl.when(in_range(j))
            def _():
                wait_read(j, b)

                @pl.when(j >= NBUF)
                def _():
                    wait_write(j - NBUF, b)

                pack(b)
                start_write(j, b)

                @pl.when(in_range(j + NBUF))
                def _():
                    start_read(j + NBUF, b)

        return carry

    lax.fori_loop(0, -(-RB_PER_W // NBUF), outer, 0)

    def drain(j, carry):
        @pl.when(in_range(j) & jnp.logical_not(in_range(j + NBUF)))
        def _():
            wait_write(j, j % NBUF)
        return carry

    lax.fori_loop(RB_PER_W - 2 * NBUF, RB_PER_W + NBUF, drain, 0)


@functools.partial(
    pl.kernel,
    mesh=_mesh,
    out_type=jax.ShapeDtypeStruct((B, D_MODEL), jnp.float32),
    compiler_params=_tiled,
    scratch_types=[
        pltpu.VMEM((PER_W * K,), jnp.int32),
        pltpu.VMEM((NBUF, K), jnp.int32),
        pltpu.VMEM((NBUF, K + 16), jnp.int32),
        pltpu.VMEM((NBUF, K, 128), jnp.float32),
        pltpu.VMEM((NBUF, K, D_MODEL), jnp.float32),
        pltpu.SemaphoreType.DMA((NBUF,)),
        pltpu.SemaphoreType.DMA((NBUF,)),
    ],
)
def _embed(x_hbm, scr_hbm, out_hbm, idx_v, idxh_v, par_v, rows_v, obuf_v,
           gsem, wsem):
    wid = lax.axis_index("s") * 2 + lax.axis_index("c")
    row0 = wid * PER_W * K

    pltpu.sync_copy(x_hbm.at[wid], idx_v)

    def prep(t, b):
        for g in range(K // 16):
            v = idx_v[pl.ds(t * K + g * 16, 16)]
            sl = pl.ds(g * 16, 16)
            idxh_v[b, sl] = lax.shift_right_logical(v, 1)
            par_v[b, sl] = lax.bitwise_and(v, 1) * D_MODEL

    def start_gather(b):
        pltpu.async_copy(scr_hbm.at[idxh_v.at[b]], rows_v.at[b], gsem.at[b])

    def wait_gather(b):
        pltpu.make_async_copy(
            scr_hbm.at[idxh_v.at[b]], rows_v.at[b], gsem.at[b]
        ).wait()

    def out_slice(t):
        return out_hbm.at[pl.ds(row0 + t * K, K), :]

    def start_write(t, b):
        pltpu.async_copy(obuf_v.at[b], out_slice(t), wsem.at[b])

    def wait_write(t, b):
        pltpu.make_async_copy(obuf_v.at[b], out_slice(t), wsem.at[b]).wait()

    def scale_chunk(b):
        def rbody(r, carry):
            po = par_v[b, pl.ds(r, 16)][0]
            for q in range(D_MODEL // 16):
                obuf_v[b, r, pl.ds(q * 16, 16)] = (
                    rows_v[b, r, pl.ds(po + q * 16, 16)] * SCALE
                )
            return carry

        lax.fori_loop(0, K, rbody, 0, unroll=2)

    for b in range(NBUF):
        prep(b, b)
        start_gather(b)

    def outer(grp, carry):
        for b in range(NBUF):
            t = grp * NBUF + b
            wait_gather(b)

            @pl.when(t >= NBUF)
            def _():
                wait_write(t - NBUF, b)

            scale_chunk(b)
            start_write(t, b)

            @pl.when(t + NBUF < PER_W)
            def _():
                prep(t + NBUF, b)
                start_gather(b)

        return carry

    lax.fori_loop(0, PER_W // NBUF, outer, 0)

    for b in range(NBUF):
        wait_write(PER_W - NBUF + b, b)


@jax.jit
def kernel(x, lut):
    scr = _repack(lut)
    xw = x.reshape(NW, PER_W * K)
    out2 = _embed(xw, scr)
    return out2.reshape(4096, 200, D_MODEL)


# speedup vs baseline: 2.3160x; 1.4806x over previous
import functools
import math

import jax
import jax.numpy as jnp
from jax import lax
from jax.experimental import pallas as pl
from jax.experimental.pallas import tpu as pltpu
from jax.experimental.pallas import tpu_sc as plsc

D_MODEL = 64
VOCAB = 1000000
ROWS = 4096
COLS = 200
B = ROWS * COLS
NW = 32
PER_W = B // NW
K = 128
NCHUNK = PER_W // K
NBUF = 4
SCALE = math.sqrt(D_MODEL)

_mesh = plsc.VectorSubcoreMesh(core_axis_name="c", subcore_axis_name="s")


@functools.partial(
    pl.kernel,
    mesh=_mesh,
    out_type=jax.ShapeDtypeStruct((B, D_MODEL), jnp.float32),
    compiler_params=pltpu.CompilerParams(use_tc_tiling_on_sc=False),
    scratch_types=[
        pltpu.VMEM((NCHUNK, K), jnp.int32),
        pltpu.VMEM((NBUF, K, D_MODEL), jnp.float32),
        pltpu.SemaphoreType.DMA((NBUF,)),
        pltpu.SemaphoreType.DMA((NBUF,)),
    ],
)
def _embed(x_hbm, lut_hbm, out_hbm, idx_v, rows_v, gsem, wsem):
    wid = lax.axis_index("s") * 2 + lax.axis_index("c")
    row_base = wid * PER_W

    pltpu.sync_copy(x_hbm.at[wid], idx_v)

    def start_gather(c, b):
        pltpu.async_copy(lut_hbm.at[idx_v.at[c]], rows_v.at[b], gsem.at[b])

    def wait_gather(c, b):
        pltpu.make_async_copy(
            lut_hbm.at[idx_v.at[c]], rows_v.at[b], gsem.at[b]
        ).wait()

    def start_write(c, b):
        pltpu.async_copy(
            rows_v.at[b], out_hbm.at[pl.ds(row_base + c * K, K)], wsem.at[b]
        )

    def wait_write(c, b):
        pltpu.make_async_copy(
            rows_v.at[b], out_hbm.at[pl.ds(row_base + c * K, K)], wsem.at[b]
        ).wait()

    for b in range(NBUF):
        start_gather(b, b)

    def outer(g, carry):
        for b in range(NBUF):
            c = g * NBUF + b
            wait_gather(c, b)

            def scale_row(r, carry2):
                for j in range(D_MODEL // 16):
                    sl = pl.ds(j * 16, 16)
                    rows_v[b, r, sl] = rows_v[b, r, sl] * SCALE
                return carry2

            lax.fori_loop(0, K, scale_row, 0, unroll=2)
            start_write(c, b)

            @pl.when(c + NBUF < NCHUNK)
            def _():
                wait_write(c, b)
                start_gather(c + NBUF, b)

        return carry

    lax.fori_loop(0, NCHUNK // NBUF, outer, 0)

    for b in range(NBUF):
        wait_write(NCHUNK - NBUF + b, b)


@jax.jit
def kernel(x, lut):
    xf = x.reshape(NW, NCHUNK, K)
    out = _embed(xf, lut)
    return out.reshape(ROWS, COLS, D_MODEL)
